# baseline (speedup 1.0000x reference)
import functools

import jax
import jax.numpy as jnp
from jax import lax
from jax.experimental import pallas as pl
from jax.experimental.pallas import tpu as pltpu
from jax.experimental.pallas import tpu_sc as plsc

NC = 2
NS = 16
NW = NC * NS
L = 16
BATCH = 16384
EMB = 16
CH = 128
ROWS = BATCH // CH
NCH = BATCH // (NW * CH)


def _sc_body(uidx, ridx, uemb, ubias, vemb, vbias, part_out, ubrb_out,
             uidx_v, ridx_v, urows_v, vrows_v, ub_v, rb_v, ubrb_v, part_v,
             sem):
    wid = lax.axis_index("s") * NC + lax.axis_index("c")
    base = wid * NCH
    pltpu.sync_copy(uidx.at[pl.ds(base, NCH)], uidx_v)
    pltpu.sync_copy(ridx.at[pl.ds(base, NCH)], ridx_v)
    copies = []
    for j in range(NCH):
        copies.append(pltpu.async_copy(uemb.at[uidx_v.at[j]], urows_v.at[j], sem))
        copies.append(pltpu.async_copy(vemb.at[ridx_v.at[j]], vrows_v.at[j], sem))
        copies.append(pltpu.async_copy(ubias.at[uidx_v.at[j]], ub_v.at[j], sem))
        copies.append(pltpu.async_copy(vbias.at[ridx_v.at[j]], rb_v.at[j], sem))
    for cp in copies:
        cp.wait()

    def row_body(r, acc):
        for j in range(NCH):
            acc = acc + urows_v[j, r, :] * vrows_v[j, r, :]
        return acc

    acc = lax.fori_loop(0, CH, row_body, jnp.zeros((L,), jnp.float32))
    part_v[...] = acc
    pltpu.sync_copy(part_v, part_out.at[wid])

    for j in range(NCH):
        for i in range(CH // L):
            sl = pl.ds(i * L, L)
            ubrb_v[j, sl] = ub_v[j, sl] + rb_v[j, sl]
    pltpu.sync_copy(ubrb_v, ubrb_out.at[pl.ds(base, NCH)])


_sc_gather_dot = pl.kernel(
    _sc_body,
    out_type=(
        jax.ShapeDtypeStruct((NW, L), jnp.float32),
        jax.ShapeDtypeStruct((ROWS, CH), jnp.float32),
    ),
    mesh=plsc.VectorSubcoreMesh(core_axis_name="c", subcore_axis_name="s"),
    scratch_types=[
        pltpu.VMEM((NCH, CH), jnp.int32),
        pltpu.VMEM((NCH, CH), jnp.int32),
        pltpu.VMEM((NCH, CH, EMB), jnp.float32),
        pltpu.VMEM((NCH, CH, EMB), jnp.float32),
        pltpu.VMEM((NCH, CH), jnp.float32),
        pltpu.VMEM((NCH, CH), jnp.float32),
        pltpu.VMEM((NCH, CH), jnp.float32),
        pltpu.VMEM((L,), jnp.float32),
        pltpu.SemaphoreType.DMA,
    ],
    compiler_params=pltpu.CompilerParams(use_tc_tiling_on_sc=False),
)


def _combine_body(part_ref, ubrb_ref, out_ref):
    s = jnp.sum(part_ref[...])
    out_ref[...] = jax.nn.sigmoid(s + ubrb_ref[...])


_combine = pl.pallas_call(
    _combine_body,
    out_shape=jax.ShapeDtypeStruct((ROWS, CH), jnp.float32),
)


@jax.jit
def kernel(inputs, user_embedding, user_bias, movie_embedding, movie_bias):
    uidx = inputs[:, 0].astype(jnp.int32).reshape(ROWS, CH)
    ridx = inputs[:, 1].astype(jnp.int32).reshape(ROWS, CH)
    part, ubrb = _sc_gather_dot(
        uidx, ridx,
        user_embedding, user_bias.reshape(-1),
        movie_embedding, movie_bias.reshape(-1),
    )
    return _combine(part, ubrb).reshape(BATCH, 1)
